# baseline (speedup 1.0000x reference)
import jax
import jax.numpy as jnp
from jax.experimental import pallas as pl
from jax.experimental.pallas import tpu as pltpu

N = 4096
D = 256
BR = 128
NB = N // BR


def _mega_kernel(x0_ref, t_ref, s_ref, o_ref, sb_ref, xb_ref, yb_ref, xf_ref, acc_ref):
    q = pl.program_id(0)
    i = pl.program_id(1)
    rows = pl.ds(i * BR, BR)

    @pl.when(q == 0)
    def _init():
        blk = x0_ref[...]
        xf_ref[rows, :] = blk
        acc_ref[rows, :] = blk
        xb_ref[rows, :] = blk.astype(jnp.bfloat16)

    @pl.when(q % 2 == 1)
    def _t_phase():
        yb_ref[rows, :] = jnp.dot(
            t_ref[...].astype(jnp.bfloat16),
            xb_ref[...],
            preferred_element_type=jnp.float32,
        ).astype(jnp.bfloat16)

    @pl.when(q == 2)
    def _s_load():
        sb_ref[rows, :] = s_ref[...].astype(jnp.bfloat16)

    @pl.when((q == 2) | (q == 4) | (q == 6))
    def _s_phase():
        st = sb_ref[rows, :]
        d = jnp.dot(st, yb_ref[...], preferred_element_type=jnp.float32)
        xn = d + xf_ref[rows, :]
        an = acc_ref[rows, :] + xn

        @pl.when(q != 6)
        def _():
            xf_ref[rows, :] = xn
            xb_ref[rows, :] = xn.astype(jnp.bfloat16)
            acc_ref[rows, :] = an

        @pl.when(q == 6)
        def _():
            o_ref[...] = 0.25 * an


def _x0_idx(q, i):
    return (jnp.where(q == 0, i, NB - 1), 0)


def _t_idx(q, i):
    return (jnp.where(q % 2 == 1, i, jnp.where(q == 0, 0, NB - 1)), 0)


def _s_idx(q, i):
    return (jnp.where(q == 2, i, jnp.where(q < 2, 0, NB - 1)), 0)


def _o_idx(q, i):
    return (jnp.where(q == 6, i, 0), 0)


def kernel(pois_embs, HG_poi_src, HG_poi_tar):
    return pl.pallas_call(
        _mega_kernel,
        grid=(7, NB),
        in_specs=[
            pl.BlockSpec((BR, D), _x0_idx),
            pl.BlockSpec((BR, N), _t_idx),
            pl.BlockSpec((BR, N), _s_idx),
        ],
        out_specs=pl.BlockSpec((BR, D), _o_idx),
        out_shape=jax.ShapeDtypeStruct((N, D), jnp.float32),
        scratch_shapes=[
            pltpu.VMEM((N, N), jnp.bfloat16),
            pltpu.VMEM((N, D), jnp.bfloat16),
            pltpu.VMEM((N, D), jnp.bfloat16),
            pltpu.VMEM((N, D), jnp.float32),
            pltpu.VMEM((N, D), jnp.float32),
        ],
        compiler_params=pltpu.CompilerParams(
            dimension_semantics=("arbitrary", "arbitrary"),
        ),
    )(pois_embs, HG_poi_tar, HG_poi_src)
